# baseline (speedup 1.0000x reference)
import functools

import jax
import jax.numpy as jnp
from jax import lax
from jax.experimental import pallas as pl
from jax.experimental.pallas import tpu as pltpu
from jax.experimental.pallas import tpu_sc as plsc

N = 10000
E = 160000
POS_DIM = 6
ISL = 6
STEPS = 2
LATENT = 128
MP_STEPS = 10
RADIUS = 0.015
INV_R = 1.0 / RADIUS

N_PAD = 10240
E_PAD = 163840
NODE_BLK = 1024
EDGE_BLK = 2048

_INTERPRET = False


def _ln(x, g, b):
    m = jnp.mean(x, axis=-1, keepdims=True)
    v = jnp.mean((x - m) * (x - m), axis=-1, keepdims=True)
    return (x - m) * jax.lax.rsqrt(v + 1e-5) * g + b


def _dot(x, w):
    return jnp.dot(x, w, preferred_element_type=jnp.float32)


def _node_enc_body(cur_ref, typ_ref, ctx_ref, temb_ref,
                   w0_ref, b0_ref, w1_ref, b1_ref, w2_ref, b2_ref,
                   g_ref, be_ref, wb_ref, wc_ref,
                   hv_ref, t1_ref, t2_ref):
    x = cur_ref[...]
    vel = x[:, 6:] - x[:, :30]
    recent = x[:, 30:]
    d_low = recent * INV_R
    d_up = (1.0 - recent) * INV_R
    boundary = jnp.clip(jnp.concatenate([d_low, d_up], axis=1), -1.0, 1.0)
    t = typ_ref[0, 0, :]
    ids = jax.lax.broadcasted_iota(jnp.int32, (t.shape[0], 9), 1)
    onehot = (t[:, None] == ids).astype(jnp.float32)
    emb = _dot(onehot, temb_ref[...])
    ctx = jnp.broadcast_to(ctx_ref[...], (t.shape[0], 6))
    nf = jnp.concatenate([vel, boundary, emb, ctx], axis=1)
    h = jax.nn.relu(_dot(nf, w0_ref[...]) + b0_ref[...])
    h = jax.nn.relu(_dot(h, w1_ref[...]) + b1_ref[...])
    h = _dot(h, w2_ref[...]) + b2_ref[...]
    h = _ln(h, g_ref[...], be_ref[...])
    hv_ref[...] = h
    t1_ref[...] = _dot(h, wb_ref[...])
    t2_ref[...] = _dot(h, wc_ref[...])


def _edge_enc_body(r1_ref, r2_ref,
                   w0_ref, b0_ref, w1_ref, b1_ref, w2_ref, b2_ref,
                   g_ref, be_ref, he_ref):
    rel = (r1_ref[...] - r2_ref[...]) * INV_R
    d2 = jnp.sum(rel * rel, axis=1, keepdims=True)
    dist = jnp.sqrt(d2)
    lane = jax.lax.broadcasted_iota(jnp.int32, rel.shape, 1)
    ef = jnp.where(lane == 6, dist, rel)
    h = jax.nn.relu(_dot(ef, w0_ref[...]) + b0_ref[...])
    h = jax.nn.relu(_dot(h, w1_ref[...]) + b1_ref[...])
    h = _dot(h, w2_ref[...]) + b2_ref[...]
    he_ref[...] = _ln(h, g_ref[...], be_ref[...])


def _edge_mp_body(he_ref, g1_ref, g2_ref,
                  w0a_ref, b0_ref, w1_ref, b1_ref, w2_ref, b2_ref,
                  g_ref, be_ref, en_ref, heo_ref):
    he = he_ref[...]
    x = jax.nn.relu(_dot(he, w0a_ref[...]) + g1_ref[...] + g2_ref[...]
                    + b0_ref[...])
    x = jax.nn.relu(_dot(x, w1_ref[...]) + b1_ref[...])
    x = _dot(x, w2_ref[...]) + b2_ref[...]
    en = _ln(x, g_ref[...], be_ref[...])
    en_ref[...] = en
    heo_ref[...] = he + en


def _node_mp_body(hv_ref, p0_ref, p1_ref,
                  wa_ref, wg_ref, b0_ref, w1_ref, b1_ref, w2_ref, b2_ref,
                  g_ref, be_ref, wb_ref, wc_ref,
                  hvo_ref, t1_ref, t2_ref):
    hv = hv_ref[...]
    agg = p0_ref[...] + p1_ref[...]
    x = jax.nn.relu(_dot(hv, wa_ref[...]) + _dot(agg, wg_ref[...])
                    + b0_ref[...])
    x = jax.nn.relu(_dot(x, w1_ref[...]) + b1_ref[...])
    x = _dot(x, w2_ref[...]) + b2_ref[...]
    hv_new = hv + _ln(x, g_ref[...], be_ref[...])
    hvo_ref[...] = hv_new
    if t1_ref is not None:
        t1_ref[...] = _dot(hv_new, wb_ref[...])
        t2_ref[...] = _dot(hv_new, wc_ref[...])


def _decoder_body(hv_ref, cur_ref, gt_ref, kin_ref,
                  w0_ref, b0_ref, w1_ref, b1_ref, w2_ref, b2_ref,
                  pred_ref, loss_ref):
    i = pl.program_id(0)
    h = jax.nn.relu(_dot(hv_ref[...], w0_ref[...]) + b0_ref[...])
    h = jax.nn.relu(_dot(h, w1_ref[...]) + b1_ref[...])
    acc = _dot(h, w2_ref[...]) + b2_ref[...]
    x = cur_ref[...]
    recent = x[:, 30:]
    prev = x[:, 24:30]
    nxt = recent + (recent - prev) + acc
    kin = kin_ref[...]
    gt = gt_ref[...]
    pred = kin * gt + (1.0 - kin) * nxt
    pred_ref[...] = pred
    rows = jax.lax.broadcasted_iota(jnp.int32, pred.shape, 0) + i * NODE_BLK
    diff = jnp.where(rows < N, pred - gt, 0.0)
    part = jnp.sum(diff * diff, keepdims=True).reshape(1, 1)

    @pl.when(i == 0)
    def _():
        loss_ref[...] = jnp.zeros_like(part)

    loss_ref[...] += part


def _full(shape):
    return pl.BlockSpec(shape, lambda i: tuple(0 for _ in shape))


def _rows(blk, width):
    return pl.BlockSpec((blk, width), lambda i: (i, 0))


_F32 = jnp.float32


def _node_enc_call(cur36, typ3d, ctx, temb, enc, ln, wb, wc):
    (w0, b0), (w1, b1), (w2, b2) = enc
    g, be = ln
    grid = N_PAD // NODE_BLK
    specs = [
        _rows(NODE_BLK, 36),
        pl.BlockSpec((1, 1, NODE_BLK), lambda i: (i, 0, 0)),
        _full((1, 6)), _full((9, 16)),
        _full((64, 128)), _full((1, 128)), _full((128, 128)), _full((1, 128)),
        _full((128, 128)), _full((1, 128)), _full((1, 128)), _full((1, 128)),
        _full((128, 128)), _full((128, 128)),
    ]
    outs = [jax.ShapeDtypeStruct((N_PAD, 128), _F32)] * 3
    out_specs = [_rows(NODE_BLK, 128)] * 3
    return pl.pallas_call(
        _node_enc_body, grid=(grid,), in_specs=specs, out_specs=out_specs,
        out_shape=outs, interpret=_INTERPRET,
    )(cur36, typ3d, ctx, temb, w0, b0[None], w1, b1[None], w2, b2[None],
      g[None], be[None], wb, wc)


def _edge_enc_call(r1, r2, w016, enc, ln):
    (_, b0), (w1, b1), (w2, b2) = enc
    g, be = ln
    grid = E_PAD // EDGE_BLK
    specs = [
        _rows(EDGE_BLK, 128), _rows(EDGE_BLK, 128),
        _full((128, 128)), _full((1, 128)), _full((128, 128)), _full((1, 128)),
        _full((128, 128)), _full((1, 128)), _full((1, 128)), _full((1, 128)),
    ]
    return pl.pallas_call(
        _edge_enc_body, grid=(grid,), in_specs=specs,
        out_specs=_rows(EDGE_BLK, 128),
        out_shape=jax.ShapeDtypeStruct((E_PAD, 128), _F32),
        interpret=_INTERPRET,
    )(r1, r2, w016, b0[None], w1, b1[None], w2, b2[None], g[None], be[None])


def _edge_mp_call(he, g1, g2, blk):
    (w0, b0), (w1, b1), (w2, b2) = blk["edge_mlp"]
    g, be = blk["edge_ln"]
    w0a = w0[:128]
    grid = E_PAD // EDGE_BLK
    specs = [
        _rows(EDGE_BLK, 128), _rows(EDGE_BLK, 128), _rows(EDGE_BLK, 128),
        _full((128, 128)), _full((1, 128)), _full((128, 128)), _full((1, 128)),
        _full((128, 128)), _full((1, 128)), _full((1, 128)), _full((1, 128)),
    ]
    outs = [jax.ShapeDtypeStruct((E_PAD, 128), _F32)] * 2
    return pl.pallas_call(
        _edge_mp_body, grid=(grid,), in_specs=specs,
        out_specs=[_rows(EDGE_BLK, 128)] * 2, out_shape=outs,
        interpret=_INTERPRET,
    )(he, g1, g2, w0a, b0[None], w1, b1[None], w2, b2[None], g[None], be[None])


def _node_mp_call(hv, p0, p1, blk, wb, wc):
    (w0, b0), (w1, b1), (w2, b2) = blk["node_mlp"]
    g, be = blk["node_ln"]
    wa, wg = w0[:128], w0[128:]
    grid = N_PAD // NODE_BLK
    with_tables = wb is not None
    specs = [
        _rows(NODE_BLK, 128), _rows(NODE_BLK, 128), _rows(NODE_BLK, 128),
        _full((128, 128)), _full((128, 128)), _full((1, 128)),
        _full((128, 128)), _full((1, 128)), _full((128, 128)), _full((1, 128)),
        _full((1, 128)), _full((1, 128)),
    ]
    args = [hv, p0, p1, wa, wg, b0[None], w1, b1[None], w2, b2[None],
            g[None], be[None]]
    if with_tables:
        specs += [_full((128, 128)), _full((128, 128))]
        args += [wb, wc]
        outs = [jax.ShapeDtypeStruct((N_PAD, 128), _F32)] * 3
        out_specs = [_rows(NODE_BLK, 128)] * 3
        body = _node_mp_body
    else:
        outs = [jax.ShapeDtypeStruct((N_PAD, 128), _F32)]
        out_specs = [_rows(NODE_BLK, 128)]

        def body(hv_ref, p0_ref, p1_ref, wa_ref, wg_ref, b0_ref, w1_ref,
                 b1_ref, w2_ref, b2_ref, g_ref, be_ref, hvo_ref):
            _node_mp_body(hv_ref, p0_ref, p1_ref, wa_ref, wg_ref, b0_ref,
                          w1_ref, b1_ref, w2_ref, b2_ref, g_ref, be_ref,
                          None, None, hvo_ref, None, None)

    return pl.pallas_call(
        body, grid=(grid,), in_specs=specs, out_specs=out_specs,
        out_shape=outs, interpret=_INTERPRET,
    )(*args)


def _decoder_call(hv, cur36, gt, kin, dec):
    (w0, b0), (w1, b1), (w2, b2) = dec
    grid = N_PAD // NODE_BLK
    specs = [
        _rows(NODE_BLK, 128), _rows(NODE_BLK, 36), _rows(NODE_BLK, 6),
        _rows(NODE_BLK, 1),
        _full((128, 128)), _full((1, 128)), _full((128, 128)), _full((1, 128)),
        _full((128, 6)), _full((1, 6)),
    ]
    outs = [jax.ShapeDtypeStruct((N_PAD, 6), _F32),
            jax.ShapeDtypeStruct((1, 1), _F32)]
    out_specs = [_rows(NODE_BLK, 6), _full((1, 1))]
    return pl.pallas_call(
        _decoder_body, grid=(grid,), in_specs=specs, out_specs=out_specs,
        out_shape=outs, interpret=_INTERPRET,
    )(hv, cur36, gt, kin, w0, b0[None], w1, b1[None], w2, b2[None])


SC_NC = 2
SC_NS = 16
SC_NW = SC_NC * SC_NS
SC_CHUNK = 128
E_PER_W = E_PAD // SC_NW
N_CHUNKS = E_PER_W // SC_CHUNK
ACC_PER_S = N_PAD // SC_NS


@functools.lru_cache(maxsize=None)
def _sc_gather_kernel(d):
    mesh = plsc.VectorSubcoreMesh(core_axis_name="c", subcore_axis_name="s")

    @functools.partial(
        pl.kernel, mesh=mesh,
        out_type=[jax.ShapeDtypeStruct((E_PAD, d), jnp.float32)] * 2,
        scratch_types=[
            pltpu.VMEM((E_PER_W,), jnp.int32),
            pltpu.VMEM((E_PER_W,), jnp.int32),
            pltpu.VMEM((3, SC_CHUNK, d), jnp.float32),
            pltpu.VMEM((3, SC_CHUNK, d), jnp.float32),
            pltpu.SemaphoreType.DMA,
            pltpu.SemaphoreType.DMA,
        ],
    )
    def k(t1_hbm, t2_hbm, i1_hbm, i2_hbm, g1_hbm, g2_hbm,
          i1_v, i2_v, b1_v, b2_v, sem1, sem2):
        wid = lax.axis_index("s") * SC_NC + lax.axis_index("c")
        base = wid * E_PER_W
        pltpu.sync_copy(i1_hbm.at[pl.ds(base, E_PER_W)], i1_v)
        pltpu.sync_copy(i2_hbm.at[pl.ds(base, E_PER_W)], i2_v)

        def run_table(t_hbm, i_v, b_v, g_hbm, sem, k_depth):
            def body(i, _):
                j0 = i * k_depth
                cps = []
                for s in range(k_depth):
                    off = (j0 + s) * SC_CHUNK
                    cps.append(pltpu.async_copy(
                        t_hbm.at[i_v.at[pl.ds(off, SC_CHUNK)]], b_v.at[s],
                        sem))
                for s in range(k_depth):
                    off = (j0 + s) * SC_CHUNK
                    cps[s].wait()
                    pltpu.sync_copy(b_v.at[s],
                                    g_hbm.at[pl.ds(base + off, SC_CHUNK)])
                return 0

            lax.fori_loop(0, N_CHUNKS // k_depth, body, 0)

        run_table(t1_hbm, i1_v, b1_v, g1_hbm, sem1, 3)
        run_table(t2_hbm, i2_v, b2_v, g2_hbm, sem2, 3)

    return k


def _sc_scatter_kernel():
    mesh = plsc.VectorSubcoreMesh(core_axis_name="c", subcore_axis_name="s")

    @functools.partial(
        pl.kernel, mesh=mesh,
        out_type=jax.ShapeDtypeStruct((SC_NC, N_PAD, LATENT), jnp.float32),
        scratch_types=[
            pltpu.VMEM((N_CHUNKS, SC_CHUNK), jnp.int32),
            pltpu.VMEM((2, SC_CHUNK, LATENT), jnp.float32),
            pltpu.VMEM_SHARED((N_PAD, LATENT), jnp.float32),
            pltpu.SemaphoreType.DMA,
        ],
    )
    def k(en_hbm, idx_hbm, zeros_hbm, out_hbm, idx_v, buf_v, acc_sh, sem):
        cid = lax.axis_index("c")
        sid = lax.axis_index("s")
        wid = sid * SC_NC + cid
        base = wid * E_PER_W
        pltpu.sync_copy(zeros_hbm, acc_sh.at[pl.ds(sid * ACC_PER_S,
                                                   ACC_PER_S)])
        pltpu.sync_copy(idx_hbm.at[wid], idx_v)
        plsc.subcore_barrier()

        def body(i, _):
            j0 = i * 2
            cps = []
            for s in range(2):
                cps.append(pltpu.async_copy(
                    en_hbm.at[pl.ds(base + (j0 + s) * SC_CHUNK, SC_CHUNK)],
                    buf_v.at[s], sem))
            for s in range(2):
                cps[s].wait()
                pltpu.sync_copy(buf_v.at[s], acc_sh.at[idx_v.at[j0 + s]],
                                add=True)
            return 0

        lax.fori_loop(0, N_CHUNKS // 2, body, 0)
        plsc.subcore_barrier()
        pltpu.sync_copy(acc_sh.at[pl.ds(sid * ACC_PER_S, ACC_PER_S)],
                        out_hbm.at[cid, pl.ds(sid * ACC_PER_S, ACC_PER_S)])

    return k


_SC_SCATTER = None


def _gather_rows(t1, t2, idx1, idx2):
    g1, g2 = _sc_gather_kernel(t1.shape[1])(t1, t2, idx1, idx2)
    return g1, g2


def _scatter_partials(en, dst3d, zeros_blk):
    global _SC_SCATTER
    if _SC_SCATTER is None:
        _SC_SCATTER = _sc_scatter_kernel()
    parts = _SC_SCATTER(en, dst3d, zeros_blk)
    return parts[0], parts[1]


def _predict_step(cur, typ3d, kin, ctx, gt_step, src_i, dst_i, dst3d,
                  zeros_blk, params):
    cur36 = cur.reshape(N_PAD, ISL * POS_DIM)
    recent128 = jnp.pad(cur[:, -1], ((0, 0), (0, 128 - POS_DIM)))

    r1, r2 = _gather_rows(recent128, recent128, src_i, dst_i)
    w0128 = jnp.pad(params["edge_enc"][0][0], ((0, 128 - 7), (0, 0)))
    he = _edge_enc_call(r1, r2, w0128, params["edge_enc"],
                        params["edge_enc_ln"])

    pb0 = params["proc"][0]
    wb0, wc0 = pb0["edge_mlp"][0][0][128:256], pb0["edge_mlp"][0][0][256:]
    hv, t1, t2 = _node_enc_call(cur36, typ3d, ctx, params["type_emb"],
                                params["node_enc"], params["node_enc_ln"],
                                wb0, wc0)

    for i, blk in enumerate(params["proc"]):
        g1, g2 = _gather_rows(t1, t2, src_i, dst_i)
        en, he = _edge_mp_call(he, g1, g2, blk)
        p0, p1 = _scatter_partials(en, dst3d, zeros_blk)
        if i + 1 < MP_STEPS:
            nb = params["proc"][i + 1]
            wbn = nb["edge_mlp"][0][0][128:256]
            wcn = nb["edge_mlp"][0][0][256:]
            hv, t1, t2 = _node_mp_call(hv, p0, p1, blk, wbn, wcn)
        else:
            (hv,) = _node_mp_call(hv, p0, p1, blk, None, None)

    pred, loss = _decoder_call(hv, cur36, gt_step, kin, params["decoder"])
    return pred, loss


def kernel(position, n_particles_per_example, particle_type, step_context,
           edge_index, params):
    del n_particles_per_example
    position = position.astype(jnp.float32)
    src = edge_index[0].astype(jnp.int32)
    dst = edge_index[1].astype(jnp.int32)
    pad_e = E_PAD - E
    src_i = jnp.pad(src, (0, pad_e))
    dst_i = jnp.pad(dst, (0, pad_e))
    dst3d = jnp.pad(dst, (0, pad_e), constant_values=N).reshape(
        SC_NW, N_CHUNKS, SC_CHUNK)
    zeros_blk = jnp.zeros((ACC_PER_S, LATENT), jnp.float32)

    typ = particle_type.astype(jnp.int32)
    typ_pad = jnp.pad(typ, (0, N_PAD - N), constant_values=-1)
    typ3d = typ_pad.reshape(N_PAD // NODE_BLK, 1, NODE_BLK)
    kin = (typ_pad == 3).astype(jnp.float32)[:, None]

    cur = jnp.pad(position[:, :ISL], ((0, N_PAD - N), (0, 0), (0, 0)))
    gt = jnp.pad(position[:, ISL:ISL + STEPS], ((0, N_PAD - N), (0, 0), (0, 0)))
    ctx = step_context.astype(jnp.float32)

    preds = []
    loss = jnp.float32(0.0)
    for step in range(STEPS):
        pred, lpart = _predict_step(cur, typ3d, kin, ctx, gt[:, step],
                                    src_i, dst_i, dst3d, zeros_blk, params)
        preds.append(pred[:N])
        loss = loss + lpart[0, 0]
        cur = jnp.concatenate([cur[:, 1:], pred[:, None, :]], axis=1)

    predictions = jnp.stack(preds)
    gt_p = jnp.transpose(gt[:N], (1, 0, 2))
    return (loss, predictions, gt_p)


# speedup vs baseline: 1.0657x; 1.0657x over previous
import functools

import jax
import jax.numpy as jnp
from jax import lax
from jax.experimental import pallas as pl
from jax.experimental.pallas import tpu as pltpu
from jax.experimental.pallas import tpu_sc as plsc

N = 10000
E = 160000
POS_DIM = 6
ISL = 6
STEPS = 2
LATENT = 128
MP_STEPS = 10
RADIUS = 0.015
INV_R = 1.0 / RADIUS

N_PAD = 10240
E_PAD = 163840
NODE_BLK = 1024
EDGE_BLK = 2048

_INTERPRET = False


def _ln(x, g, b):
    m = jnp.mean(x, axis=-1, keepdims=True)
    v = jnp.mean((x - m) * (x - m), axis=-1, keepdims=True)
    return (x - m) * jax.lax.rsqrt(v + 1e-5) * g + b


def _dot(x, w):
    return jnp.dot(x, w, preferred_element_type=jnp.float32)


def _node_enc_body(cur_ref, typ_ref, ctx_ref, temb_ref,
                   w0_ref, b0_ref, w1_ref, b1_ref, w2_ref, b2_ref,
                   g_ref, be_ref, wb_ref, wc_ref,
                   hv_ref, t1_ref, t2_ref):
    x = cur_ref[...]
    vel = x[:, 6:] - x[:, :30]
    recent = x[:, 30:]
    d_low = recent * INV_R
    d_up = (1.0 - recent) * INV_R
    boundary = jnp.clip(jnp.concatenate([d_low, d_up], axis=1), -1.0, 1.0)
    t = typ_ref[0, 0, :]
    ids = jax.lax.broadcasted_iota(jnp.int32, (t.shape[0], 9), 1)
    onehot = (t[:, None] == ids).astype(jnp.float32)
    emb = _dot(onehot, temb_ref[...])
    ctx = jnp.broadcast_to(ctx_ref[...], (t.shape[0], 6))
    nf = jnp.concatenate([vel, boundary, emb, ctx], axis=1)
    h = jax.nn.relu(_dot(nf, w0_ref[...]) + b0_ref[...])
    h = jax.nn.relu(_dot(h, w1_ref[...]) + b1_ref[...])
    h = _dot(h, w2_ref[...]) + b2_ref[...]
    h = _ln(h, g_ref[...], be_ref[...])
    hv_ref[...] = h
    t1_ref[...] = _dot(h, wb_ref[...])
    t2_ref[...] = _dot(h, wc_ref[...])


def _edge_enc_body(r1_ref, r2_ref,
                   w0_ref, b0_ref, w1_ref, b1_ref, w2_ref, b2_ref,
                   g_ref, be_ref, he_ref):
    rel = (r1_ref[...] - r2_ref[...]) * INV_R
    d2 = jnp.sum(rel * rel, axis=1, keepdims=True)
    dist = jnp.sqrt(d2)
    lane = jax.lax.broadcasted_iota(jnp.int32, rel.shape, 1)
    ef = jnp.where(lane == 6, dist, rel)
    h = jax.nn.relu(_dot(ef, w0_ref[...]) + b0_ref[...])
    h = jax.nn.relu(_dot(h, w1_ref[...]) + b1_ref[...])
    h = _dot(h, w2_ref[...]) + b2_ref[...]
    he_ref[...] = _ln(h, g_ref[...], be_ref[...])


def _edge_mp_body(he_ref, g1_ref, g2_ref,
                  w0a_ref, b0_ref, w1_ref, b1_ref, w2_ref, b2_ref,
                  g_ref, be_ref, en_ref, heo_ref):
    he = he_ref[...]
    x = jax.nn.relu(_dot(he, w0a_ref[...]) + g1_ref[...] + g2_ref[...]
                    + b0_ref[...])
    x = jax.nn.relu(_dot(x, w1_ref[...]) + b1_ref[...])
    x = _dot(x, w2_ref[...]) + b2_ref[...]
    en = _ln(x, g_ref[...], be_ref[...])
    en_ref[...] = en
    heo_ref[...] = he + en


def _node_mp_body(hv_ref, p0_ref, p1_ref,
                  wa_ref, wg_ref, b0_ref, w1_ref, b1_ref, w2_ref, b2_ref,
                  g_ref, be_ref, wb_ref, wc_ref,
                  hvo_ref, t1_ref, t2_ref):
    hv = hv_ref[...]
    agg = p0_ref[...] + p1_ref[...]
    x = jax.nn.relu(_dot(hv, wa_ref[...]) + _dot(agg, wg_ref[...])
                    + b0_ref[...])
    x = jax.nn.relu(_dot(x, w1_ref[...]) + b1_ref[...])
    x = _dot(x, w2_ref[...]) + b2_ref[...]
    hv_new = hv + _ln(x, g_ref[...], be_ref[...])
    hvo_ref[...] = hv_new
    if t1_ref is not None:
        t1_ref[...] = _dot(hv_new, wb_ref[...])
        t2_ref[...] = _dot(hv_new, wc_ref[...])


def _decoder_body(hv_ref, cur_ref, gt_ref, kin_ref,
                  w0_ref, b0_ref, w1_ref, b1_ref, w2_ref, b2_ref,
                  pred_ref, loss_ref):
    i = pl.program_id(0)
    h = jax.nn.relu(_dot(hv_ref[...], w0_ref[...]) + b0_ref[...])
    h = jax.nn.relu(_dot(h, w1_ref[...]) + b1_ref[...])
    acc = _dot(h, w2_ref[...]) + b2_ref[...]
    x = cur_ref[...]
    recent = x[:, 30:]
    prev = x[:, 24:30]
    nxt = recent + (recent - prev) + acc
    kin = kin_ref[...]
    gt = gt_ref[...]
    pred = kin * gt + (1.0 - kin) * nxt
    pred_ref[...] = pred
    rows = jax.lax.broadcasted_iota(jnp.int32, pred.shape, 0) + i * NODE_BLK
    diff = jnp.where(rows < N, pred - gt, 0.0)
    part = jnp.sum(diff * diff, keepdims=True).reshape(1, 1)

    @pl.when(i == 0)
    def _():
        loss_ref[...] = jnp.zeros_like(part)

    loss_ref[...] += part


def _full(shape):
    return pl.BlockSpec(shape, lambda i: tuple(0 for _ in shape))


def _rows(blk, width):
    return pl.BlockSpec((blk, width), lambda i: (i, 0))


_F32 = jnp.float32


def _node_enc_call(cur36, typ3d, ctx, temb, enc, ln, wb, wc):
    (w0, b0), (w1, b1), (w2, b2) = enc
    g, be = ln
    grid = N_PAD // NODE_BLK
    specs = [
        _rows(NODE_BLK, 36),
        pl.BlockSpec((1, 1, NODE_BLK), lambda i: (i, 0, 0)),
        _full((1, 6)), _full((9, 16)),
        _full((64, 128)), _full((1, 128)), _full((128, 128)), _full((1, 128)),
        _full((128, 128)), _full((1, 128)), _full((1, 128)), _full((1, 128)),
        _full((128, 128)), _full((128, 128)),
    ]
    outs = [jax.ShapeDtypeStruct((N_PAD, 128), _F32)] * 3
    out_specs = [_rows(NODE_BLK, 128)] * 3
    return pl.pallas_call(
        _node_enc_body, grid=(grid,), in_specs=specs, out_specs=out_specs,
        out_shape=outs, interpret=_INTERPRET,
    )(cur36, typ3d, ctx, temb, w0, b0[None], w1, b1[None], w2, b2[None],
      g[None], be[None], wb, wc)


def _edge_enc_call(r1, r2, w016, enc, ln):
    (_, b0), (w1, b1), (w2, b2) = enc
    g, be = ln
    grid = E_PAD // EDGE_BLK
    specs = [
        _rows(EDGE_BLK, 128), _rows(EDGE_BLK, 128),
        _full((128, 128)), _full((1, 128)), _full((128, 128)), _full((1, 128)),
        _full((128, 128)), _full((1, 128)), _full((1, 128)), _full((1, 128)),
    ]
    return pl.pallas_call(
        _edge_enc_body, grid=(grid,), in_specs=specs,
        out_specs=_rows(EDGE_BLK, 128),
        out_shape=jax.ShapeDtypeStruct((E_PAD, 128), _F32),
        interpret=_INTERPRET,
    )(r1, r2, w016, b0[None], w1, b1[None], w2, b2[None], g[None], be[None])


def _edge_mp_call(he, g1, g2, blk):
    (w0, b0), (w1, b1), (w2, b2) = blk["edge_mlp"]
    g, be = blk["edge_ln"]
    w0a = w0[:128]
    grid = E_PAD // EDGE_BLK
    specs = [
        _rows(EDGE_BLK, 128), _rows(EDGE_BLK, 128), _rows(EDGE_BLK, 128),
        _full((128, 128)), _full((1, 128)), _full((128, 128)), _full((1, 128)),
        _full((128, 128)), _full((1, 128)), _full((1, 128)), _full((1, 128)),
    ]
    outs = [jax.ShapeDtypeStruct((E_PAD, 128), _F32)] * 2
    return pl.pallas_call(
        _edge_mp_body, grid=(grid,), in_specs=specs,
        out_specs=[_rows(EDGE_BLK, 128)] * 2, out_shape=outs,
        interpret=_INTERPRET,
    )(he, g1, g2, w0a, b0[None], w1, b1[None], w2, b2[None], g[None], be[None])


def _node_mp_call(hv, p0, p1, blk, wb, wc):
    (w0, b0), (w1, b1), (w2, b2) = blk["node_mlp"]
    g, be = blk["node_ln"]
    wa, wg = w0[:128], w0[128:]
    grid = N_PAD // NODE_BLK
    with_tables = wb is not None
    specs = [
        _rows(NODE_BLK, 128), _rows(NODE_BLK, 128), _rows(NODE_BLK, 128),
        _full((128, 128)), _full((128, 128)), _full((1, 128)),
        _full((128, 128)), _full((1, 128)), _full((128, 128)), _full((1, 128)),
        _full((1, 128)), _full((1, 128)),
    ]
    args = [hv, p0, p1, wa, wg, b0[None], w1, b1[None], w2, b2[None],
            g[None], be[None]]
    if with_tables:
        specs += [_full((128, 128)), _full((128, 128))]
        args += [wb, wc]
        outs = [jax.ShapeDtypeStruct((N_PAD, 128), _F32)] * 3
        out_specs = [_rows(NODE_BLK, 128)] * 3
        body = _node_mp_body
    else:
        outs = [jax.ShapeDtypeStruct((N_PAD, 128), _F32)]
        out_specs = [_rows(NODE_BLK, 128)]

        def body(hv_ref, p0_ref, p1_ref, wa_ref, wg_ref, b0_ref, w1_ref,
                 b1_ref, w2_ref, b2_ref, g_ref, be_ref, hvo_ref):
            _node_mp_body(hv_ref, p0_ref, p1_ref, wa_ref, wg_ref, b0_ref,
                          w1_ref, b1_ref, w2_ref, b2_ref, g_ref, be_ref,
                          None, None, hvo_ref, None, None)

    return pl.pallas_call(
        body, grid=(grid,), in_specs=specs, out_specs=out_specs,
        out_shape=outs, interpret=_INTERPRET,
    )(*args)


def _decoder_call(hv, cur36, gt, kin, dec):
    (w0, b0), (w1, b1), (w2, b2) = dec
    grid = N_PAD // NODE_BLK
    specs = [
        _rows(NODE_BLK, 128), _rows(NODE_BLK, 36), _rows(NODE_BLK, 6),
        _rows(NODE_BLK, 1),
        _full((128, 128)), _full((1, 128)), _full((128, 128)), _full((1, 128)),
        _full((128, 6)), _full((1, 6)),
    ]
    outs = [jax.ShapeDtypeStruct((N_PAD, 6), _F32),
            jax.ShapeDtypeStruct((1, 1), _F32)]
    out_specs = [_rows(NODE_BLK, 6), _full((1, 1))]
    return pl.pallas_call(
        _decoder_body, grid=(grid,), in_specs=specs, out_specs=out_specs,
        out_shape=outs, interpret=_INTERPRET,
    )(hv, cur36, gt, kin, w0, b0[None], w1, b1[None], w2, b2[None])


SC_NC = 2
SC_NS = 16
SC_NW = SC_NC * SC_NS
SC_CHUNK = 128
E_PER_W = E_PAD // SC_NW
N_CHUNKS = E_PER_W // SC_CHUNK
ACC_PER_S = N_PAD // SC_NS


@functools.lru_cache(maxsize=None)
def _sc_gather_kernel(d):
    mesh = plsc.VectorSubcoreMesh(core_axis_name="c", subcore_axis_name="s")

    @functools.partial(
        pl.kernel, mesh=mesh,
        out_type=[jax.ShapeDtypeStruct((E_PAD, d), jnp.float32)] * 2,
        scratch_types=[
            pltpu.VMEM((E_PER_W,), jnp.int32),
            pltpu.VMEM((E_PER_W,), jnp.int32),
            pltpu.VMEM((SC_CHUNK, d), jnp.float32),
            pltpu.VMEM((SC_CHUNK, d), jnp.float32),
            pltpu.SemaphoreType.DMA,
            pltpu.SemaphoreType.DMA,
        ],
    )
    def k(t1_hbm, t2_hbm, i1_hbm, i2_hbm, g1_hbm, g2_hbm,
          i1_v, i2_v, b1_v, b2_v, sem1, sem2):
        wid = lax.axis_index("s") * SC_NC + lax.axis_index("c")
        base = wid * E_PER_W
        pltpu.sync_copy(i1_hbm.at[pl.ds(base, E_PER_W)], i1_v)
        pltpu.sync_copy(i2_hbm.at[pl.ds(base, E_PER_W)], i2_v)

        def body(j, _):
            off = j * SC_CHUNK
            cp1 = pltpu.async_copy(
                t1_hbm.at[i1_v.at[pl.ds(off, SC_CHUNK)]], b1_v, sem1)
            cp2 = pltpu.async_copy(
                t2_hbm.at[i2_v.at[pl.ds(off, SC_CHUNK)]], b2_v, sem2)
            cp1.wait()
            pltpu.sync_copy(b1_v, g1_hbm.at[pl.ds(base + off, SC_CHUNK)])
            cp2.wait()
            pltpu.sync_copy(b2_v, g2_hbm.at[pl.ds(base + off, SC_CHUNK)])
            return 0

        lax.fori_loop(0, N_CHUNKS, body, 0)

    return k


def _sc_scatter_kernel():
    mesh = plsc.VectorSubcoreMesh(core_axis_name="c", subcore_axis_name="s")

    @functools.partial(
        pl.kernel, mesh=mesh,
        out_type=jax.ShapeDtypeStruct((SC_NC, N_PAD, LATENT), jnp.float32),
        scratch_types=[
            pltpu.VMEM((N_CHUNKS, SC_CHUNK), jnp.int32),
            pltpu.VMEM((SC_CHUNK, LATENT), jnp.float32),
            pltpu.VMEM_SHARED((N_PAD, LATENT), jnp.float32),
            pltpu.SemaphoreType.DMA,
        ],
    )
    def k(en_hbm, idx_hbm, zeros_hbm, out_hbm, idx_v, buf_v, acc_sh, sem):
        cid = lax.axis_index("c")
        sid = lax.axis_index("s")
        wid = sid * SC_NC + cid
        base = wid * E_PER_W
        pltpu.sync_copy(zeros_hbm, acc_sh.at[pl.ds(sid * ACC_PER_S,
                                                   ACC_PER_S)])
        pltpu.sync_copy(idx_hbm.at[wid], idx_v)
        plsc.subcore_barrier()

        def body(j, _):
            cp = pltpu.async_copy(
                en_hbm.at[pl.ds(base + j * SC_CHUNK, SC_CHUNK)], buf_v, sem)
            cp.wait()
            pltpu.sync_copy(buf_v, acc_sh.at[idx_v.at[j]], add=True)
            return 0

        lax.fori_loop(0, N_CHUNKS, body, 0)
        plsc.subcore_barrier()
        pltpu.sync_copy(acc_sh.at[pl.ds(sid * ACC_PER_S, ACC_PER_S)],
                        out_hbm.at[cid, pl.ds(sid * ACC_PER_S, ACC_PER_S)])

    return k


_SC_SCATTER = None


def _gather_rows(t1, t2, idx1, idx2):
    g1, g2 = _sc_gather_kernel(t1.shape[1])(t1, t2, idx1, idx2)
    return g1, g2


def _scatter_partials(en, dst3d, zeros_blk):
    global _SC_SCATTER
    if _SC_SCATTER is None:
        _SC_SCATTER = _sc_scatter_kernel()
    parts = _SC_SCATTER(en, dst3d, zeros_blk)
    return parts[0], parts[1]


def _predict_step(cur, typ3d, kin, ctx, gt_step, src_i, dst_i, dst3d,
                  zeros_blk, params):
    cur36 = cur.reshape(N_PAD, ISL * POS_DIM)
    recent128 = jnp.pad(cur[:, -1], ((0, 0), (0, 128 - POS_DIM)))

    r1, r2 = _gather_rows(recent128, recent128, src_i, dst_i)
    w0128 = jnp.pad(params["edge_enc"][0][0], ((0, 128 - 7), (0, 0)))
    he = _edge_enc_call(r1, r2, w0128, params["edge_enc"],
                        params["edge_enc_ln"])

    pb0 = params["proc"][0]
    wb0, wc0 = pb0["edge_mlp"][0][0][128:256], pb0["edge_mlp"][0][0][256:]
    hv, t1, t2 = _node_enc_call(cur36, typ3d, ctx, params["type_emb"],
                                params["node_enc"], params["node_enc_ln"],
                                wb0, wc0)

    for i, blk in enumerate(params["proc"]):
        g1, g2 = _gather_rows(t1, t2, src_i, dst_i)
        en, he = _edge_mp_call(he, g1, g2, blk)
        p0, p1 = _scatter_partials(en, dst3d, zeros_blk)
        if i + 1 < MP_STEPS:
            nb = params["proc"][i + 1]
            wbn = nb["edge_mlp"][0][0][128:256]
            wcn = nb["edge_mlp"][0][0][256:]
            hv, t1, t2 = _node_mp_call(hv, p0, p1, blk, wbn, wcn)
        else:
            (hv,) = _node_mp_call(hv, p0, p1, blk, None, None)

    pred, loss = _decoder_call(hv, cur36, gt_step, kin, params["decoder"])
    return pred, loss


def kernel(position, n_particles_per_example, particle_type, step_context,
           edge_index, params):
    del n_particles_per_example
    position = position.astype(jnp.float32)
    order = jnp.argsort(edge_index[1])
    src = edge_index[0, order].astype(jnp.int32)
    dst = edge_index[1, order].astype(jnp.int32)
    pad_e = E_PAD - E
    src_i = jnp.pad(src, (0, pad_e))
    dst_i = jnp.pad(dst, (0, pad_e))
    dst3d = jnp.pad(dst, (0, pad_e), constant_values=N).reshape(
        SC_NW, N_CHUNKS, SC_CHUNK)
    zeros_blk = jnp.zeros((ACC_PER_S, LATENT), jnp.float32)

    typ = particle_type.astype(jnp.int32)
    typ_pad = jnp.pad(typ, (0, N_PAD - N), constant_values=-1)
    typ3d = typ_pad.reshape(N_PAD // NODE_BLK, 1, NODE_BLK)
    kin = (typ_pad == 3).astype(jnp.float32)[:, None]

    cur = jnp.pad(position[:, :ISL], ((0, N_PAD - N), (0, 0), (0, 0)))
    gt = jnp.pad(position[:, ISL:ISL + STEPS], ((0, N_PAD - N), (0, 0), (0, 0)))
    ctx = step_context.astype(jnp.float32)

    preds = []
    loss = jnp.float32(0.0)
    for step in range(STEPS):
        pred, lpart = _predict_step(cur, typ3d, kin, ctx, gt[:, step],
                                    src_i, dst_i, dst3d, zeros_blk, params)
        preds.append(pred[:N])
        loss = loss + lpart[0, 0]
        cur = jnp.concatenate([cur[:, 1:], pred[:, None, :]], axis=1)

    predictions = jnp.stack(preds)
    gt_p = jnp.transpose(gt[:N], (1, 0, 2))
    return (loss, predictions, gt_p)


# speedup vs baseline: 1.3072x; 1.2267x over previous
import functools

import jax
import jax.numpy as jnp
from jax import lax
from jax.experimental import pallas as pl
from jax.experimental.pallas import tpu as pltpu
from jax.experimental.pallas import tpu_sc as plsc

N = 10000
E = 160000
POS_DIM = 6
ISL = 6
STEPS = 2
LATENT = 128
MP_STEPS = 10
RADIUS = 0.015
INV_R = 1.0 / RADIUS

N_PAD = 10240
E_PAD = 163840
NODE_BLK = 1024
EDGE_BLK = 2048

_INTERPRET = False


def _ln(x, g, b):
    m = jnp.mean(x, axis=-1, keepdims=True)
    v = jnp.mean((x - m) * (x - m), axis=-1, keepdims=True)
    return (x - m) * jax.lax.rsqrt(v + 1e-5) * g + b


def _dot(x, w):
    return jnp.dot(x, w, preferred_element_type=jnp.float32)


def _node_enc_body(cur_ref, typ_ref, ctx_ref, temb_ref,
                   w0_ref, b0_ref, w1_ref, b1_ref, w2_ref, b2_ref,
                   g_ref, be_ref, wb_ref, wc_ref,
                   hv_ref, t1_ref, t2_ref):
    x = cur_ref[...]
    vel = x[:, 6:] - x[:, :30]
    recent = x[:, 30:]
    d_low = recent * INV_R
    d_up = (1.0 - recent) * INV_R
    boundary = jnp.clip(jnp.concatenate([d_low, d_up], axis=1), -1.0, 1.0)
    t = typ_ref[0, 0, :]
    ids = jax.lax.broadcasted_iota(jnp.int32, (t.shape[0], 9), 1)
    onehot = (t[:, None] == ids).astype(jnp.float32)
    emb = _dot(onehot, temb_ref[...])
    ctx = jnp.broadcast_to(ctx_ref[...], (t.shape[0], 6))
    nf = jnp.concatenate([vel, boundary, emb, ctx], axis=1)
    h = jax.nn.relu(_dot(nf, w0_ref[...]) + b0_ref[...])
    h = jax.nn.relu(_dot(h, w1_ref[...]) + b1_ref[...])
    h = _dot(h, w2_ref[...]) + b2_ref[...]
    h = _ln(h, g_ref[...], be_ref[...])
    hv_ref[...] = h
    t1_ref[...] = _dot(h, wb_ref[...])
    t2_ref[...] = _dot(h, wc_ref[...])


def _edge_enc_body(r1_ref, r2_ref,
                   w0_ref, b0_ref, w1_ref, b1_ref, w2_ref, b2_ref,
                   g_ref, be_ref, he_ref):
    rel = (r1_ref[...] - r2_ref[...]) * INV_R
    d2 = jnp.sum(rel * rel, axis=1, keepdims=True)
    dist = jnp.sqrt(d2)
    lane = jax.lax.broadcasted_iota(jnp.int32, rel.shape, 1)
    ef = jnp.where(lane == 6, dist, rel)
    h = jax.nn.relu(_dot(ef, w0_ref[...]) + b0_ref[...])
    h = jax.nn.relu(_dot(h, w1_ref[...]) + b1_ref[...])
    h = _dot(h, w2_ref[...]) + b2_ref[...]
    he_ref[...] = _ln(h, g_ref[...], be_ref[...])


def _edge_mp_body(he_ref, g1_ref, g2_ref,
                  w0a_ref, b0_ref, w1_ref, b1_ref, w2_ref, b2_ref,
                  g_ref, be_ref, en_ref, heo_ref):
    he = he_ref[...]
    x = jax.nn.relu(_dot(he, w0a_ref[...]) + g1_ref[...] + g2_ref[...]
                    + b0_ref[...])
    x = jax.nn.relu(_dot(x, w1_ref[...]) + b1_ref[...])
    x = _dot(x, w2_ref[...]) + b2_ref[...]
    en = _ln(x, g_ref[...], be_ref[...])
    en_ref[...] = en
    heo_ref[...] = he + en


def _node_mp_body(hv_ref, p0_ref, p1_ref, p2_ref, p3_ref,
                  wa_ref, wg_ref, b0_ref, w1_ref, b1_ref, w2_ref, b2_ref,
                  g_ref, be_ref, wb_ref, wc_ref,
                  hvo_ref, t1_ref, t2_ref):
    hv = hv_ref[...]
    agg = (p0_ref[...] + p1_ref[...]) + (p2_ref[...] + p3_ref[...])
    x = jax.nn.relu(_dot(hv, wa_ref[...]) + _dot(agg, wg_ref[...])
                    + b0_ref[...])
    x = jax.nn.relu(_dot(x, w1_ref[...]) + b1_ref[...])
    x = _dot(x, w2_ref[...]) + b2_ref[...]
    hv_new = hv + _ln(x, g_ref[...], be_ref[...])
    hvo_ref[...] = hv_new
    if t1_ref is not None:
        t1_ref[...] = _dot(hv_new, wb_ref[...])
        t2_ref[...] = _dot(hv_new, wc_ref[...])


def _decoder_body(hv_ref, cur_ref, gt_ref, kin_ref,
                  w0_ref, b0_ref, w1_ref, b1_ref, w2_ref, b2_ref,
                  pred_ref, loss_ref):
    i = pl.program_id(0)
    h = jax.nn.relu(_dot(hv_ref[...], w0_ref[...]) + b0_ref[...])
    h = jax.nn.relu(_dot(h, w1_ref[...]) + b1_ref[...])
    acc = _dot(h, w2_ref[...]) + b2_ref[...]
    x = cur_ref[...]
    recent = x[:, 30:]
    prev = x[:, 24:30]
    nxt = recent + (recent - prev) + acc
    kin = kin_ref[...]
    gt = gt_ref[...]
    pred = kin * gt + (1.0 - kin) * nxt
    pred_ref[...] = pred
    rows = jax.lax.broadcasted_iota(jnp.int32, pred.shape, 0) + i * NODE_BLK
    diff = jnp.where(rows < N, pred - gt, 0.0)
    part = jnp.sum(diff * diff, keepdims=True).reshape(1, 1)

    @pl.when(i == 0)
    def _():
        loss_ref[...] = jnp.zeros_like(part)

    loss_ref[...] += part


def _full(shape):
    return pl.BlockSpec(shape, lambda i: tuple(0 for _ in shape))


def _rows(blk, width):
    return pl.BlockSpec((blk, width), lambda i: (i, 0))


_F32 = jnp.float32


def _node_enc_call(cur36, typ3d, ctx, temb, enc, ln, wb, wc):
    (w0, b0), (w1, b1), (w2, b2) = enc
    g, be = ln
    grid = N_PAD // NODE_BLK
    specs = [
        _rows(NODE_BLK, 36),
        pl.BlockSpec((1, 1, NODE_BLK), lambda i: (i, 0, 0)),
        _full((1, 6)), _full((9, 16)),
        _full((64, 128)), _full((1, 128)), _full((128, 128)), _full((1, 128)),
        _full((128, 128)), _full((1, 128)), _full((1, 128)), _full((1, 128)),
        _full((128, 128)), _full((128, 128)),
    ]
    outs = [jax.ShapeDtypeStruct((N_PAD, 128), _F32)] * 3
    out_specs = [_rows(NODE_BLK, 128)] * 3
    return pl.pallas_call(
        _node_enc_body, grid=(grid,), in_specs=specs, out_specs=out_specs,
        out_shape=outs, interpret=_INTERPRET,
    )(cur36, typ3d, ctx, temb, w0, b0[None], w1, b1[None], w2, b2[None],
      g[None], be[None], wb, wc)


def _edge_enc_call(r1, r2, w016, enc, ln):
    (_, b0), (w1, b1), (w2, b2) = enc
    g, be = ln
    grid = r1.shape[0] // EDGE_BLK
    specs = [
        _rows(EDGE_BLK, 128), _rows(EDGE_BLK, 128),
        _full((128, 128)), _full((1, 128)), _full((128, 128)), _full((1, 128)),
        _full((128, 128)), _full((1, 128)), _full((1, 128)), _full((1, 128)),
    ]
    return pl.pallas_call(
        _edge_enc_body, grid=(grid,), in_specs=specs,
        out_specs=_rows(EDGE_BLK, 128),
        out_shape=jax.ShapeDtypeStruct((r1.shape[0], 128), _F32),
        interpret=_INTERPRET,
    )(r1, r2, w016, b0[None], w1, b1[None], w2, b2[None], g[None], be[None])


def _edge_mp_call(he, g1, g2, blk):
    (w0, b0), (w1, b1), (w2, b2) = blk["edge_mlp"]
    g, be = blk["edge_ln"]
    w0a = w0[:128]
    grid = he.shape[0] // EDGE_BLK
    specs = [
        _rows(EDGE_BLK, 128), _rows(EDGE_BLK, 128), _rows(EDGE_BLK, 128),
        _full((128, 128)), _full((1, 128)), _full((128, 128)), _full((1, 128)),
        _full((128, 128)), _full((1, 128)), _full((1, 128)), _full((1, 128)),
    ]
    outs = [jax.ShapeDtypeStruct((he.shape[0], 128), _F32)] * 2
    return pl.pallas_call(
        _edge_mp_body, grid=(grid,), in_specs=specs,
        out_specs=[_rows(EDGE_BLK, 128)] * 2, out_shape=outs,
        interpret=_INTERPRET,
    )(he, g1, g2, w0a, b0[None], w1, b1[None], w2, b2[None], g[None], be[None])


def _node_mp_call(hv, parts, blk, wb, wc):
    (w0, b0), (w1, b1), (w2, b2) = blk["node_mlp"]
    g, be = blk["node_ln"]
    wa, wg = w0[:128], w0[128:]
    grid = N_PAD // NODE_BLK
    with_tables = wb is not None
    p0, p1, p2, p3 = parts
    specs = [
        _rows(NODE_BLK, 128), _rows(NODE_BLK, 128), _rows(NODE_BLK, 128),
        _rows(NODE_BLK, 128), _rows(NODE_BLK, 128),
        _full((128, 128)), _full((128, 128)), _full((1, 128)),
        _full((128, 128)), _full((1, 128)), _full((128, 128)), _full((1, 128)),
        _full((1, 128)), _full((1, 128)),
    ]
    args = [hv, p0, p1, p2, p3, wa, wg, b0[None], w1, b1[None], w2, b2[None],
            g[None], be[None]]
    if with_tables:
        specs += [_full((128, 128)), _full((128, 128))]
        args += [wb, wc]
        outs = [jax.ShapeDtypeStruct((N_PAD, 128), _F32)] * 3
        out_specs = [_rows(NODE_BLK, 128)] * 3
        body = _node_mp_body
    else:
        outs = [jax.ShapeDtypeStruct((N_PAD, 128), _F32)]
        out_specs = [_rows(NODE_BLK, 128)]

        def body(hv_ref, p0_ref, p1_ref, p2_ref, p3_ref, wa_ref, wg_ref,
                 b0_ref, w1_ref, b1_ref, w2_ref, b2_ref, g_ref, be_ref,
                 hvo_ref):
            _node_mp_body(hv_ref, p0_ref, p1_ref, p2_ref, p3_ref, wa_ref,
                          wg_ref, b0_ref, w1_ref, b1_ref, w2_ref, b2_ref,
                          g_ref, be_ref, None, None, hvo_ref, None, None)

    return pl.pallas_call(
        body, grid=(grid,), in_specs=specs, out_specs=out_specs,
        out_shape=outs, interpret=_INTERPRET,
    )(*args)


def _decoder_call(hv, cur36, gt, kin, dec):
    (w0, b0), (w1, b1), (w2, b2) = dec
    grid = N_PAD // NODE_BLK
    specs = [
        _rows(NODE_BLK, 128), _rows(NODE_BLK, 36), _rows(NODE_BLK, 6),
        _rows(NODE_BLK, 1),
        _full((128, 128)), _full((1, 128)), _full((128, 128)), _full((1, 128)),
        _full((128, 6)), _full((1, 6)),
    ]
    outs = [jax.ShapeDtypeStruct((N_PAD, 6), _F32),
            jax.ShapeDtypeStruct((1, 1), _F32)]
    out_specs = [_rows(NODE_BLK, 6), _full((1, 1))]
    return pl.pallas_call(
        _decoder_body, grid=(grid,), in_specs=specs, out_specs=out_specs,
        out_shape=outs, interpret=_INTERPRET,
    )(hv, cur36, gt, kin, w0, b0[None], w1, b1[None], w2, b2[None])


SC_NC = 2
SC_NS = 16
SC_NW = SC_NC * SC_NS
SC_CHUNK = 128
E_HALF = E_PAD // 2
ACC_PER_S = N_PAD // SC_NS


@functools.lru_cache(maxsize=None)
def _sc_gather_kernel(d, n_rows):
    per_w = n_rows // SC_NW
    n_chunks = per_w // SC_CHUNK
    mesh = plsc.VectorSubcoreMesh(core_axis_name="c", subcore_axis_name="s")

    @functools.partial(
        pl.kernel, mesh=mesh,
        out_type=[jax.ShapeDtypeStruct((n_rows, d), jnp.float32)] * 2,
        scratch_types=[
            pltpu.VMEM((per_w,), jnp.int32),
            pltpu.VMEM((per_w,), jnp.int32),
            pltpu.VMEM((SC_CHUNK, d), jnp.float32),
            pltpu.VMEM((SC_CHUNK, d), jnp.float32),
            pltpu.SemaphoreType.DMA,
            pltpu.SemaphoreType.DMA,
        ],
    )
    def k(t1_hbm, t2_hbm, i1_hbm, i2_hbm, g1_hbm, g2_hbm,
          i1_v, i2_v, b1_v, b2_v, sem1, sem2):
        wid = lax.axis_index("s") * SC_NC + lax.axis_index("c")
        base = wid * per_w
        pltpu.sync_copy(i1_hbm.at[pl.ds(base, per_w)], i1_v)
        pltpu.sync_copy(i2_hbm.at[pl.ds(base, per_w)], i2_v)

        def body(j, _):
            off = j * SC_CHUNK
            cp1 = pltpu.async_copy(
                t1_hbm.at[i1_v.at[pl.ds(off, SC_CHUNK)]], b1_v, sem1)
            cp2 = pltpu.async_copy(
                t2_hbm.at[i2_v.at[pl.ds(off, SC_CHUNK)]], b2_v, sem2)
            cp1.wait()
            pltpu.sync_copy(b1_v, g1_hbm.at[pl.ds(base + off, SC_CHUNK)])
            cp2.wait()
            pltpu.sync_copy(b2_v, g2_hbm.at[pl.ds(base + off, SC_CHUNK)])
            return 0

        lax.fori_loop(0, n_chunks, body, 0)

    return k


@functools.lru_cache(maxsize=None)
def _sc_scatter_kernel(n_rows):
    per_w = n_rows // SC_NW
    n_chunks = per_w // SC_CHUNK
    mesh = plsc.VectorSubcoreMesh(core_axis_name="c", subcore_axis_name="s")

    @functools.partial(
        pl.kernel, mesh=mesh,
        out_type=jax.ShapeDtypeStruct((SC_NC, N_PAD, LATENT), jnp.float32),
        scratch_types=[
            pltpu.VMEM((n_chunks, SC_CHUNK), jnp.int32),
            pltpu.VMEM((SC_CHUNK, LATENT), jnp.float32),
            pltpu.VMEM_SHARED((N_PAD, LATENT), jnp.float32),
            pltpu.SemaphoreType.DMA,
        ],
    )
    def k(en_hbm, idx_hbm, zeros_hbm, out_hbm, idx_v, buf_v, acc_sh, sem):
        cid = lax.axis_index("c")
        sid = lax.axis_index("s")
        wid = sid * SC_NC + cid
        base = wid * per_w
        pltpu.sync_copy(zeros_hbm, acc_sh.at[pl.ds(sid * ACC_PER_S,
                                                   ACC_PER_S)])
        pltpu.sync_copy(idx_hbm.at[wid], idx_v)
        plsc.subcore_barrier()

        def body(j, _):
            cp = pltpu.async_copy(
                en_hbm.at[pl.ds(base + j * SC_CHUNK, SC_CHUNK)], buf_v, sem)
            cp.wait()
            pltpu.sync_copy(buf_v, acc_sh.at[idx_v.at[j]], add=True)
            return 0

        lax.fori_loop(0, n_chunks, body, 0)
        plsc.subcore_barrier()
        pltpu.sync_copy(acc_sh.at[pl.ds(sid * ACC_PER_S, ACC_PER_S)],
                        out_hbm.at[cid, pl.ds(sid * ACC_PER_S, ACC_PER_S)])

    return k


def _gather_rows(t1, t2, idx1, idx2):
    g1, g2 = _sc_gather_kernel(t1.shape[1], idx1.shape[0])(t1, t2, idx1, idx2)
    return g1, g2


def _scatter_partials(en, dst3d, zeros_blk):
    parts = _sc_scatter_kernel(en.shape[0])(en, dst3d, zeros_blk)
    return parts[0], parts[1]


def _predict_step(cur, typ3d, kin, ctx, gt_step, halves, zeros_blk, params):
    cur36 = cur.reshape(N_PAD, ISL * POS_DIM)
    recent128 = jnp.pad(cur[:, -1], ((0, 0), (0, 128 - POS_DIM)))

    w0128 = jnp.pad(params["edge_enc"][0][0], ((0, 128 - 7), (0, 0)))
    he = []
    for (src_h, dst_h, _) in halves:
        r1, r2 = _gather_rows(recent128, recent128, src_h, dst_h)
        he.append(_edge_enc_call(r1, r2, w0128, params["edge_enc"],
                                 params["edge_enc_ln"]))

    pb0 = params["proc"][0]
    wb0, wc0 = pb0["edge_mlp"][0][0][128:256], pb0["edge_mlp"][0][0][256:]
    hv, t1, t2 = _node_enc_call(cur36, typ3d, ctx, params["type_emb"],
                                params["node_enc"], params["node_enc_ln"],
                                wb0, wc0)

    for i, blk in enumerate(params["proc"]):
        parts = []
        for h, (src_h, dst_h, dst3d_h) in enumerate(halves):
            g1, g2 = _gather_rows(t1, t2, src_h, dst_h)
            en, he[h] = _edge_mp_call(he[h], g1, g2, blk)
            parts.extend(_scatter_partials(en, dst3d_h, zeros_blk))
        if i + 1 < MP_STEPS:
            nb = params["proc"][i + 1]
            wbn = nb["edge_mlp"][0][0][128:256]
            wcn = nb["edge_mlp"][0][0][256:]
            hv, t1, t2 = _node_mp_call(hv, parts, blk, wbn, wcn)
        else:
            (hv,) = _node_mp_call(hv, parts, blk, None, None)

    pred, loss = _decoder_call(hv, cur36, gt_step, kin, params["decoder"])
    return pred, loss


def kernel(position, n_particles_per_example, particle_type, step_context,
           edge_index, params):
    del n_particles_per_example
    position = position.astype(jnp.float32)
    src = edge_index[0].astype(jnp.int32)
    dst = edge_index[1].astype(jnp.int32)
    pad_e = E_PAD - E
    src_i = jnp.pad(src, (0, pad_e))
    dst_i = jnp.pad(dst, (0, pad_e))
    dst_sc = jnp.pad(dst, (0, pad_e), constant_values=N)
    halves = []
    for h in range(2):
        sl = slice(h * E_HALF, (h + 1) * E_HALF)
        halves.append((src_i[sl], dst_i[sl],
                       dst_sc[sl].reshape(SC_NW, -1, SC_CHUNK)))
    zeros_blk = jnp.zeros((ACC_PER_S, LATENT), jnp.float32)

    typ = particle_type.astype(jnp.int32)
    typ_pad = jnp.pad(typ, (0, N_PAD - N), constant_values=-1)
    typ3d = typ_pad.reshape(N_PAD // NODE_BLK, 1, NODE_BLK)
    kin = (typ_pad == 3).astype(jnp.float32)[:, None]

    cur = jnp.pad(position[:, :ISL], ((0, N_PAD - N), (0, 0), (0, 0)))
    gt = jnp.pad(position[:, ISL:ISL + STEPS], ((0, N_PAD - N), (0, 0), (0, 0)))
    ctx = step_context.astype(jnp.float32)

    preds = []
    loss = jnp.float32(0.0)
    for step in range(STEPS):
        pred, lpart = _predict_step(cur, typ3d, kin, ctx, gt[:, step],
                                    halves, zeros_blk, params)
        preds.append(pred[:N])
        loss = loss + lpart[0, 0]
        cur = jnp.concatenate([cur[:, 1:], pred[:, None, :]], axis=1)

    predictions = jnp.stack(preds)
    gt_p = jnp.transpose(gt[:N], (1, 0, 2))
    return (loss, predictions, gt_p)


# speedup vs baseline: 1.3854x; 1.0598x over previous
import functools

import jax
import jax.numpy as jnp
from jax import lax
from jax.experimental import pallas as pl
from jax.experimental.pallas import tpu as pltpu
from jax.experimental.pallas import tpu_sc as plsc

N = 10000
E = 160000
POS_DIM = 6
ISL = 6
STEPS = 2
LATENT = 128
MP_STEPS = 10
RADIUS = 0.015
INV_R = 1.0 / RADIUS

N_PAD = 10240
E_PAD = 163840
NODE_BLK = 1024
EDGE_BLK = 2048

_INTERPRET = False


def _ln(x, g, b):
    m = jnp.mean(x, axis=-1, keepdims=True)
    v = jnp.mean((x - m) * (x - m), axis=-1, keepdims=True)
    return (x - m) * jax.lax.rsqrt(v + 1e-5) * g + b


def _dot(x, w):
    return jnp.dot(x, w, preferred_element_type=jnp.float32)


def _node_enc_body(cur_ref, typ_ref, ctx_ref, temb_ref,
                   w0_ref, b0_ref, w1_ref, b1_ref, w2_ref, b2_ref,
                   g_ref, be_ref, wb_ref, wc_ref,
                   hv_ref, t1_ref, t2_ref):
    x = cur_ref[...]
    vel = x[:, 6:] - x[:, :30]
    recent = x[:, 30:]
    d_low = recent * INV_R
    d_up = (1.0 - recent) * INV_R
    boundary = jnp.clip(jnp.concatenate([d_low, d_up], axis=1), -1.0, 1.0)
    t = typ_ref[0, 0, :]
    ids = jax.lax.broadcasted_iota(jnp.int32, (t.shape[0], 9), 1)
    onehot = (t[:, None] == ids).astype(jnp.float32)
    emb = _dot(onehot, temb_ref[...])
    ctx = jnp.broadcast_to(ctx_ref[...], (t.shape[0], 6))
    nf = jnp.concatenate([vel, boundary, emb, ctx], axis=1)
    h = jax.nn.relu(_dot(nf, w0_ref[...]) + b0_ref[...])
    h = jax.nn.relu(_dot(h, w1_ref[...]) + b1_ref[...])
    h = _dot(h, w2_ref[...]) + b2_ref[...]
    h = _ln(h, g_ref[...], be_ref[...])
    hv_ref[...] = h
    t1_ref[...] = _dot(h, wb_ref[...])
    t2_ref[...] = _dot(h, wc_ref[...])


def _edge_enc_body(r1_ref, r2_ref,
                   w0_ref, b0_ref, w1_ref, b1_ref, w2_ref, b2_ref,
                   g_ref, be_ref, he_ref):
    rel = (r1_ref[...] - r2_ref[...]) * INV_R
    d2 = jnp.sum(rel * rel, axis=1, keepdims=True)
    dist = jnp.sqrt(d2)
    lane = jax.lax.broadcasted_iota(jnp.int32, rel.shape, 1)
    ef = jnp.where(lane == 6, dist, rel)
    h = jax.nn.relu(_dot(ef, w0_ref[...]) + b0_ref[...])
    h = jax.nn.relu(_dot(h, w1_ref[...]) + b1_ref[...])
    h = _dot(h, w2_ref[...]) + b2_ref[...]
    he_ref[...] = _ln(h, g_ref[...], be_ref[...])


def _edge_mp_body(he_ref, g1_ref, g2_ref,
                  w0a_ref, b0_ref, w1_ref, b1_ref, w2_ref, b2_ref,
                  g_ref, be_ref, en_ref, heo_ref):
    he = he_ref[...]
    x = jax.nn.relu(_dot(he, w0a_ref[...]) + g1_ref[...] + g2_ref[...]
                    + b0_ref[...])
    x = jax.nn.relu(_dot(x, w1_ref[...]) + b1_ref[...])
    x = _dot(x, w2_ref[...]) + b2_ref[...]
    en = _ln(x, g_ref[...], be_ref[...])
    en_ref[...] = en
    heo_ref[...] = he + en


def _node_mp_body(hv_ref, p0_ref, p1_ref, p2_ref, p3_ref,
                  wa_ref, wg_ref, b0_ref, w1_ref, b1_ref, w2_ref, b2_ref,
                  g_ref, be_ref, wb_ref, wc_ref,
                  hvo_ref, t1_ref, t2_ref):
    hv = hv_ref[...]
    agg = (p0_ref[...] + p1_ref[...]) + (p2_ref[...] + p3_ref[...])
    x = jax.nn.relu(_dot(hv, wa_ref[...]) + _dot(agg, wg_ref[...])
                    + b0_ref[...])
    x = jax.nn.relu(_dot(x, w1_ref[...]) + b1_ref[...])
    x = _dot(x, w2_ref[...]) + b2_ref[...]
    hv_new = hv + _ln(x, g_ref[...], be_ref[...])
    hvo_ref[...] = hv_new
    if t1_ref is not None:
        t1_ref[...] = _dot(hv_new, wb_ref[...])
        t2_ref[...] = _dot(hv_new, wc_ref[...])


def _decoder_body(hv_ref, cur_ref, gt_ref, kin_ref,
                  w0_ref, b0_ref, w1_ref, b1_ref, w2_ref, b2_ref,
                  pred_ref, loss_ref):
    i = pl.program_id(0)
    h = jax.nn.relu(_dot(hv_ref[...], w0_ref[...]) + b0_ref[...])
    h = jax.nn.relu(_dot(h, w1_ref[...]) + b1_ref[...])
    acc = _dot(h, w2_ref[...]) + b2_ref[...]
    x = cur_ref[...]
    recent = x[:, 30:]
    prev = x[:, 24:30]
    nxt = recent + (recent - prev) + acc
    kin = kin_ref[...]
    gt = gt_ref[...]
    pred = kin * gt + (1.0 - kin) * nxt
    pred_ref[...] = pred
    rows = jax.lax.broadcasted_iota(jnp.int32, pred.shape, 0) + i * NODE_BLK
    diff = jnp.where(rows < N, pred - gt, 0.0)
    part = jnp.sum(diff * diff, keepdims=True).reshape(1, 1)

    @pl.when(i == 0)
    def _():
        loss_ref[...] = jnp.zeros_like(part)

    loss_ref[...] += part


def _full(shape):
    return pl.BlockSpec(shape, lambda i: tuple(0 for _ in shape))


def _rows(blk, width):
    return pl.BlockSpec((blk, width), lambda i: (i, 0))


_F32 = jnp.float32


def _node_enc_call(cur36, typ3d, ctx, temb, enc, ln, wb, wc):
    (w0, b0), (w1, b1), (w2, b2) = enc
    g, be = ln
    grid = N_PAD // NODE_BLK
    specs = [
        _rows(NODE_BLK, 36),
        pl.BlockSpec((1, 1, NODE_BLK), lambda i: (i, 0, 0)),
        _full((1, 6)), _full((9, 16)),
        _full((64, 128)), _full((1, 128)), _full((128, 128)), _full((1, 128)),
        _full((128, 128)), _full((1, 128)), _full((1, 128)), _full((1, 128)),
        _full((128, 128)), _full((128, 128)),
    ]
    outs = [jax.ShapeDtypeStruct((N_PAD, 128), _F32)] * 3
    out_specs = [_rows(NODE_BLK, 128)] * 3
    return pl.pallas_call(
        _node_enc_body, grid=(grid,), in_specs=specs, out_specs=out_specs,
        out_shape=outs, interpret=_INTERPRET,
    )(cur36, typ3d, ctx, temb, w0, b0[None], w1, b1[None], w2, b2[None],
      g[None], be[None], wb, wc)


def _edge_enc_call(r1, r2, w016, enc, ln):
    (_, b0), (w1, b1), (w2, b2) = enc
    g, be = ln
    grid = r1.shape[0] // EDGE_BLK
    specs = [
        _rows(EDGE_BLK, 128), _rows(EDGE_BLK, 128),
        _full((128, 128)), _full((1, 128)), _full((128, 128)), _full((1, 128)),
        _full((128, 128)), _full((1, 128)), _full((1, 128)), _full((1, 128)),
    ]
    return pl.pallas_call(
        _edge_enc_body, grid=(grid,), in_specs=specs,
        out_specs=_rows(EDGE_BLK, 128),
        out_shape=jax.ShapeDtypeStruct((r1.shape[0], 128), _F32),
        interpret=_INTERPRET,
    )(r1, r2, w016, b0[None], w1, b1[None], w2, b2[None], g[None], be[None])


def _edge_mp_call(he, g1, g2, blk):
    (w0, b0), (w1, b1), (w2, b2) = blk["edge_mlp"]
    g, be = blk["edge_ln"]
    w0a = w0[:128]
    grid = he.shape[0] // EDGE_BLK
    specs = [
        _rows(EDGE_BLK, 128), _rows(EDGE_BLK, 128), _rows(EDGE_BLK, 128),
        _full((128, 128)), _full((1, 128)), _full((128, 128)), _full((1, 128)),
        _full((128, 128)), _full((1, 128)), _full((1, 128)), _full((1, 128)),
    ]
    outs = [jax.ShapeDtypeStruct((he.shape[0], 128), _F32)] * 2
    return pl.pallas_call(
        _edge_mp_body, grid=(grid,), in_specs=specs,
        out_specs=[_rows(EDGE_BLK, 128)] * 2, out_shape=outs,
        interpret=_INTERPRET,
    )(he, g1, g2, w0a, b0[None], w1, b1[None], w2, b2[None], g[None], be[None])


def _node_mp_call(hv, parts, blk, wb, wc):
    (w0, b0), (w1, b1), (w2, b2) = blk["node_mlp"]
    g, be = blk["node_ln"]
    wa, wg = w0[:128], w0[128:]
    grid = N_PAD // NODE_BLK
    with_tables = wb is not None
    p0, p1, p2, p3 = parts
    specs = [
        _rows(NODE_BLK, 128), _rows(NODE_BLK, 128), _rows(NODE_BLK, 128),
        _rows(NODE_BLK, 128), _rows(NODE_BLK, 128),
        _full((128, 128)), _full((128, 128)), _full((1, 128)),
        _full((128, 128)), _full((1, 128)), _full((128, 128)), _full((1, 128)),
        _full((1, 128)), _full((1, 128)),
    ]
    args = [hv, p0, p1, p2, p3, wa, wg, b0[None], w1, b1[None], w2, b2[None],
            g[None], be[None]]
    if with_tables:
        specs += [_full((128, 128)), _full((128, 128))]
        args += [wb, wc]
        outs = [jax.ShapeDtypeStruct((N_PAD, 128), _F32)] * 3
        out_specs = [_rows(NODE_BLK, 128)] * 3
        body = _node_mp_body
    else:
        outs = [jax.ShapeDtypeStruct((N_PAD, 128), _F32)]
        out_specs = [_rows(NODE_BLK, 128)]

        def body(hv_ref, p0_ref, p1_ref, p2_ref, p3_ref, wa_ref, wg_ref,
                 b0_ref, w1_ref, b1_ref, w2_ref, b2_ref, g_ref, be_ref,
                 hvo_ref):
            _node_mp_body(hv_ref, p0_ref, p1_ref, p2_ref, p3_ref, wa_ref,
                          wg_ref, b0_ref, w1_ref, b1_ref, w2_ref, b2_ref,
                          g_ref, be_ref, None, None, hvo_ref, None, None)

    return pl.pallas_call(
        body, grid=(grid,), in_specs=specs, out_specs=out_specs,
        out_shape=outs, interpret=_INTERPRET,
    )(*args)


def _decoder_call(hv, cur36, gt, kin, dec):
    (w0, b0), (w1, b1), (w2, b2) = dec
    grid = N_PAD // NODE_BLK
    specs = [
        _rows(NODE_BLK, 128), _rows(NODE_BLK, 36), _rows(NODE_BLK, 6),
        _rows(NODE_BLK, 1),
        _full((128, 128)), _full((1, 128)), _full((128, 128)), _full((1, 128)),
        _full((128, 6)), _full((1, 6)),
    ]
    outs = [jax.ShapeDtypeStruct((N_PAD, 6), _F32),
            jax.ShapeDtypeStruct((1, 1), _F32)]
    out_specs = [_rows(NODE_BLK, 6), _full((1, 1))]
    return pl.pallas_call(
        _decoder_body, grid=(grid,), in_specs=specs, out_specs=out_specs,
        out_shape=outs, interpret=_INTERPRET,
    )(hv, cur36, gt, kin, w0, b0[None], w1, b1[None], w2, b2[None])


SC_NC = 2
SC_NS = 16
SC_NW = SC_NC * SC_NS
SC_CHUNK = 128
E_HALF = E_PAD // 2
ACC_PER_S = N_PAD // SC_NS


@functools.lru_cache(maxsize=None)
def _sc_gather_kernel(d, n_rows):
    per_w = n_rows // SC_NW
    n_chunks = per_w // SC_CHUNK
    mesh = plsc.VectorSubcoreMesh(core_axis_name="c", subcore_axis_name="s")

    @functools.partial(
        pl.kernel, mesh=mesh,
        out_type=[jax.ShapeDtypeStruct((n_rows, d), jnp.float32)] * 2,
        scratch_types=[
            pltpu.VMEM((per_w,), jnp.int32),
            pltpu.VMEM((per_w,), jnp.int32),
            pltpu.VMEM((SC_CHUNK, d), jnp.float32),
            pltpu.VMEM((SC_CHUNK, d), jnp.float32),
            pltpu.VMEM((SC_CHUNK, d), jnp.float32),
            pltpu.VMEM((SC_CHUNK, d), jnp.float32),
        ] + [pltpu.SemaphoreType.DMA] * 8,
    )
    def k(t1_hbm, t2_hbm, i1_hbm, i2_hbm, g1_hbm, g2_hbm,
          i1_v, i2_v, b1a, b1b, b2a, b2b,
          sg1a, sg1b, sg2a, sg2b, sw1a, sw1b, sw2a, sw2b):
        wid = lax.axis_index("s") * SC_NC + lax.axis_index("c")
        base = wid * per_w
        pltpu.sync_copy(i1_hbm.at[pl.ds(base, per_w)], i1_v)
        pltpu.sync_copy(i2_hbm.at[pl.ds(base, per_w)], i2_v)
        b1 = (b1a, b1b)
        b2 = (b2a, b2b)
        sg1 = (sg1a, sg1b)
        sg2 = (sg2a, sg2b)
        sw1 = (sw1a, sw1b)
        sw2 = (sw2a, sw2b)

        def gathers(j, s):
            off = j * SC_CHUNK
            c1 = pltpu.async_copy(
                t1_hbm.at[i1_v.at[pl.ds(off, SC_CHUNK)]], b1[s], sg1[s])
            c2 = pltpu.async_copy(
                t2_hbm.at[i2_v.at[pl.ds(off, SC_CHUNK)]], b2[s], sg2[s])
            return c1, c2

        gh = [gathers(0, 0), None]
        wh = [None, None]
        for j in range(n_chunks):
            s = j & 1
            nx = s ^ 1
            if j + 1 < n_chunks:
                if wh[nx] is not None:
                    wh[nx][0].wait()
                    wh[nx][1].wait()
                gh[nx] = gathers(j + 1, nx)
            gh[s][0].wait()
            gh[s][1].wait()
            off = base + j * SC_CHUNK
            wh[s] = (
                pltpu.async_copy(b1[s], g1_hbm.at[pl.ds(off, SC_CHUNK)],
                                 sw1[s]),
                pltpu.async_copy(b2[s], g2_hbm.at[pl.ds(off, SC_CHUNK)],
                                 sw2[s]),
            )
        for h in wh:
            if h is not None:
                h[0].wait()
                h[1].wait()

    return k


@functools.lru_cache(maxsize=None)
def _sc_scatter_kernel(n_rows):
    per_w = n_rows // SC_NW
    n_chunks = per_w // SC_CHUNK
    mesh = plsc.VectorSubcoreMesh(core_axis_name="c", subcore_axis_name="s")

    @functools.partial(
        pl.kernel, mesh=mesh,
        out_type=jax.ShapeDtypeStruct((SC_NC, N_PAD, LATENT), jnp.float32),
        scratch_types=[
            pltpu.VMEM((n_chunks, SC_CHUNK), jnp.int32),
            pltpu.VMEM((SC_CHUNK, LATENT), jnp.float32),
            pltpu.VMEM((SC_CHUNK, LATENT), jnp.float32),
            pltpu.VMEM_SHARED((N_PAD, LATENT), jnp.float32),
            pltpu.SemaphoreType.DMA,
            pltpu.SemaphoreType.DMA,
        ],
    )
    def k(en_hbm, idx_hbm, zeros_hbm, out_hbm, idx_v, bufa, bufb, acc_sh,
          sra, srb):
        cid = lax.axis_index("c")
        sid = lax.axis_index("s")
        wid = sid * SC_NC + cid
        base = wid * per_w
        pltpu.sync_copy(zeros_hbm, acc_sh.at[pl.ds(sid * ACC_PER_S,
                                                   ACC_PER_S)])
        pltpu.sync_copy(idx_hbm.at[wid], idx_v)
        plsc.subcore_barrier()
        bufs = (bufa, bufb)
        srs = (sra, srb)

        def rd(j, s):
            return pltpu.async_copy(
                en_hbm.at[pl.ds(base + j * SC_CHUNK, SC_CHUNK)], bufs[s],
                srs[s])

        h = [rd(0, 0), None]
        for j in range(n_chunks):
            s = j & 1
            if j + 1 < n_chunks:
                h[s ^ 1] = rd(j + 1, s ^ 1)
            h[s].wait()
            pltpu.sync_copy(bufs[s], acc_sh.at[idx_v.at[j]], add=True)
        plsc.subcore_barrier()
        pltpu.sync_copy(acc_sh.at[pl.ds(sid * ACC_PER_S, ACC_PER_S)],
                        out_hbm.at[cid, pl.ds(sid * ACC_PER_S, ACC_PER_S)])

    return k


def _gather_rows(t1, t2, idx1, idx2):
    g1, g2 = _sc_gather_kernel(t1.shape[1], idx1.shape[0])(t1, t2, idx1, idx2)
    return g1, g2


def _scatter_partials(en, dst3d, zeros_blk):
    parts = _sc_scatter_kernel(en.shape[0])(en, dst3d, zeros_blk)
    return parts[0], parts[1]


def _predict_step(cur, typ3d, kin, ctx, gt_step, halves, zeros_blk, params):
    cur36 = cur.reshape(N_PAD, ISL * POS_DIM)
    recent128 = jnp.pad(cur[:, -1], ((0, 0), (0, 128 - POS_DIM)))

    w0128 = jnp.pad(params["edge_enc"][0][0], ((0, 128 - 7), (0, 0)))
    he = []
    for (src_h, dst_h, _) in halves:
        r1, r2 = _gather_rows(recent128, recent128, src_h, dst_h)
        he.append(_edge_enc_call(r1, r2, w0128, params["edge_enc"],
                                 params["edge_enc_ln"]))

    pb0 = params["proc"][0]
    wb0, wc0 = pb0["edge_mlp"][0][0][128:256], pb0["edge_mlp"][0][0][256:]
    hv, t1, t2 = _node_enc_call(cur36, typ3d, ctx, params["type_emb"],
                                params["node_enc"], params["node_enc_ln"],
                                wb0, wc0)

    for i, blk in enumerate(params["proc"]):
        parts = []
        for h, (src_h, dst_h, dst3d_h) in enumerate(halves):
            g1, g2 = _gather_rows(t1, t2, src_h, dst_h)
            en, he[h] = _edge_mp_call(he[h], g1, g2, blk)
            parts.extend(_scatter_partials(en, dst3d_h, zeros_blk))
        if i + 1 < MP_STEPS:
            nb = params["proc"][i + 1]
            wbn = nb["edge_mlp"][0][0][128:256]
            wcn = nb["edge_mlp"][0][0][256:]
            hv, t1, t2 = _node_mp_call(hv, parts, blk, wbn, wcn)
        else:
            (hv,) = _node_mp_call(hv, parts, blk, None, None)

    pred, loss = _decoder_call(hv, cur36, gt_step, kin, params["decoder"])
    return pred, loss


def kernel(position, n_particles_per_example, particle_type, step_context,
           edge_index, params):
    del n_particles_per_example
    position = position.astype(jnp.float32)
    src = edge_index[0].astype(jnp.int32)
    dst = edge_index[1].astype(jnp.int32)
    pad_e = E_PAD - E
    src_i = jnp.pad(src, (0, pad_e))
    dst_i = jnp.pad(dst, (0, pad_e))
    dst_sc = jnp.pad(dst, (0, pad_e), constant_values=N)
    halves = []
    for h in range(2):
        sl = slice(h * E_HALF, (h + 1) * E_HALF)
        halves.append((src_i[sl], dst_i[sl],
                       dst_sc[sl].reshape(SC_NW, -1, SC_CHUNK)))
    zeros_blk = jnp.zeros((ACC_PER_S, LATENT), jnp.float32)

    typ = particle_type.astype(jnp.int32)
    typ_pad = jnp.pad(typ, (0, N_PAD - N), constant_values=-1)
    typ3d = typ_pad.reshape(N_PAD // NODE_BLK, 1, NODE_BLK)
    kin = (typ_pad == 3).astype(jnp.float32)[:, None]

    cur = jnp.pad(position[:, :ISL], ((0, N_PAD - N), (0, 0), (0, 0)))
    gt = jnp.pad(position[:, ISL:ISL + STEPS], ((0, N_PAD - N), (0, 0), (0, 0)))
    ctx = step_context.astype(jnp.float32)

    preds = []
    loss = jnp.float32(0.0)
    for step in range(STEPS):
        pred, lpart = _predict_step(cur, typ3d, kin, ctx, gt[:, step],
                                    halves, zeros_blk, params)
        preds.append(pred[:N])
        loss = loss + lpart[0, 0]
        cur = jnp.concatenate([cur[:, 1:], pred[:, None, :]], axis=1)

    predictions = jnp.stack(preds)
    gt_p = jnp.transpose(gt[:N], (1, 0, 2))
    return (loss, predictions, gt_p)
